# baseline (speedup 1.0000x reference)
import functools

import jax
import jax.numpy as jnp
from jax import lax
from jax.experimental import pallas as pl
from jax.experimental.pallas import tpu as pltpu
from jax.experimental.pallas import tpu_sc as plsc

_ROT = ((13, 15, 26, 6), (17, 29, 16, 24))
_K1 = 0
_K2 = 42
_K3 = _K1 ^ _K2 ^ 0x1BD11BDA
_KS = (_K1, _K2, _K3)

_NROWS = 128
_NCOLS = 100000
_SC_ROWS = 32000
_TC_ROWS = _NCOLS - _SC_ROWS
_TC_BLOCK = 3400
_XF_BLOCK = 800
_SC_TILES = 32
_ROWS_PER_TILE = _SC_ROWS // _SC_TILES

_T_THRESH = 7.213475204444817
_C_CENTRAL = (1.2533715963363647, 0.22709418833255768, 0.008377129212021828,
              -0.0014314615400508046, 5.1060102123301476e-05,
              6.718465215271863e-07)
_C_TAIL = (0.475555956363678, 0.614536702632904, 0.146858349442482,
           -0.012614135630428791)


def _rotl(v, d):
    return (v << jnp.uint32(d)) | (v >> jnp.uint32(32 - d))


def _threefry_bits(x1):
    x0 = x1
    x1 = x0 ^ _rotl(x1, 13)
    for r in _ROT[0][1:]:
        x0 = x0 + x1
        x1 = _rotl(x1, r)
        x1 = x0 ^ x1
    x0 = x0 + jnp.uint32(_KS[1])
    x1 = x1 + jnp.uint32((_KS[2] + 1) & 0xFFFFFFFF)
    for rnd in range(1, 5):
        for r in _ROT[rnd % 2]:
            x0 = x0 + x1
            x1 = _rotl(x1, r)
            x1 = x0 ^ x1
        x0 = x0 + jnp.uint32(_KS[(rnd + 1) % 3])
        x1 = x1 + jnp.uint32((_KS[(rnd + 2) % 3] + rnd + 1) & 0xFFFFFFFF)
    return x0 ^ x1


def _horner(coeffs, v):
    p = jnp.float32(coeffs[-1])
    for c in coeffs[-2::-1]:
        p = jnp.float32(c) + p * v
    return p


def _bits_to_normal(bits):
    mant = (bits >> jnp.uint32(9)) | jnp.uint32(0x3F800000)
    f = jax.lax.bitcast_convert_type(mant, jnp.float32) - jnp.float32(1.0)
    lo = jnp.float32(-0.99999994)
    hi = jnp.float32(1.0)
    u = jnp.maximum(lo, f * (hi - lo) + lo)
    t = -jnp.log2(jnp.float32(1.0) - u * u)
    q_central = _horner(_C_CENTRAL, t)
    q_tail = _horner(_C_TAIL, jnp.sqrt(t))
    q = jnp.where(t < jnp.float32(_T_THRESH), q_central, q_tail)
    return u * q


def _tc_main_kernel(o_ref):
    c0 = pl.program_id(0) * _TC_BLOCK
    shape = (_TC_BLOCK, _NROWS)
    cc = jax.lax.broadcasted_iota(jnp.uint32, shape, 0)
    rr = jax.lax.broadcasted_iota(jnp.uint32, shape, 1)
    base = jnp.uint32(c0) + jnp.uint32(_K2)
    x1 = rr * jnp.uint32(_NCOLS) + (cc + base)
    o_ref[...] = _bits_to_normal(_threefry_bits(x1))


def _tc_transform_kernel(_main_ref, bits_ref, o_ref):
    o_ref[...] = _bits_to_normal(bits_ref[...])


def _sc_bits_kernel(out_hbm, scratch, sem):
    wid = lax.axis_index("s") * 2 + lax.axis_index("c")

    @pl.when(wid < _SC_TILES)
    def _():
        row0 = wid * _ROWS_PER_TILE
        lane = lax.iota(jnp.uint32, 16)
        vecs = [(lane + jnp.uint32(16 * j)) * jnp.uint32(_NCOLS)
                for j in range(8)]

        def body(r, carry):
            c = jnp.uint32(_TC_ROWS) + jnp.uint32(row0) + jnp.uint32(r)
            base = c + jnp.uint32(_K2)
            for j in range(8):
                bits = _threefry_bits(vecs[j] + base)
                scratch[r, pl.ds(16 * j, 16)] = bits
            return carry

        lax.fori_loop(0, _ROWS_PER_TILE, body, jnp.uint32(0))
        pltpu.async_copy(
            scratch,
            out_hbm.at[pl.ds(row0, _ROWS_PER_TILE)],
            sem,
        ).wait()


def _sc_bits():
    mesh = plsc.VectorSubcoreMesh(core_axis_name="c", subcore_axis_name="s")
    return pl.kernel(
        _sc_bits_kernel,
        mesh=mesh,
        out_type=jax.ShapeDtypeStruct((_SC_ROWS, _NROWS), jnp.uint32),
        scratch_types=[
            pltpu.VMEM((_ROWS_PER_TILE, _NROWS), jnp.uint32),
            pltpu.SemaphoreType.DMA,
        ],
    )()


@functools.partial(jax.jit, static_argnames=())
def kernel(x):
    nrows, ncols = x.shape
    sc_bits = _sc_bits()

    out_main = pl.pallas_call(
        _tc_main_kernel,
        grid=(_TC_ROWS // _TC_BLOCK,),
        out_specs=pl.BlockSpec((_TC_BLOCK, _NROWS), lambda b: (b, 0)),
        out_shape=jax.ShapeDtypeStruct((_NCOLS, _NROWS), jnp.float32),
        compiler_params=pltpu.CompilerParams(
            dimension_semantics=("arbitrary",),
        ),
    )()

    out_t = pl.pallas_call(
        _tc_transform_kernel,
        grid=(_SC_ROWS // _XF_BLOCK,),
        in_specs=[
            pl.BlockSpec(memory_space=pl.ANY),
            pl.BlockSpec((_XF_BLOCK, _NROWS), lambda b: (b, 0)),
        ],
        out_specs=pl.BlockSpec((_XF_BLOCK, _NROWS),
                               lambda b: (b + _TC_ROWS // _XF_BLOCK, 0)),
        out_shape=jax.ShapeDtypeStruct((_NCOLS, _NROWS), jnp.float32),
        input_output_aliases={0: 0},
        compiler_params=pltpu.CompilerParams(
            dimension_semantics=("arbitrary",),
        ),
    )(out_main, sc_bits)
    return out_t.T


# speedup vs baseline: 1.0559x; 1.0559x over previous
import functools

import jax
import jax.numpy as jnp
from jax import lax
from jax.experimental import pallas as pl
from jax.experimental.pallas import tpu as pltpu
from jax.experimental.pallas import tpu_sc as plsc

_ROT = ((13, 15, 26, 6), (17, 29, 16, 24))
_K1 = 0
_K2 = 42
_K3 = _K1 ^ _K2 ^ 0x1BD11BDA
_KS = (_K1, _K2, _K3)

_NROWS = 128
_NCOLS = 100000
_SC_ROWS = 30000
_TC_ROWS = _NCOLS - _SC_ROWS
_TC_BLOCK = 2800
_XF_BLOCK = 10000
_SC_TILES = 30
_ROWS_PER_TILE = _SC_ROWS // _SC_TILES

_T_THRESH = 7.213475204444817
_C_CENTRAL = (1.2533715963363647, 0.22709418833255768, 0.008377129212021828,
              -0.0014314615400508046, 5.1060102123301476e-05,
              6.718465215271863e-07)
_C_TAIL = (0.475555956363678, 0.614536702632904, 0.146858349442482,
           -0.012614135630428791)


def _rotl(v, d):
    return (v << jnp.uint32(d)) | (v >> jnp.uint32(32 - d))


def _threefry_bits(x1):
    x0 = x1
    x1 = x0 ^ _rotl(x1, 13)
    for r in _ROT[0][1:]:
        x0 = x0 + x1
        x1 = _rotl(x1, r)
        x1 = x0 ^ x1
    x0 = x0 + jnp.uint32(_KS[1])
    x1 = x1 + jnp.uint32((_KS[2] + 1) & 0xFFFFFFFF)
    for rnd in range(1, 5):
        for r in _ROT[rnd % 2]:
            x0 = x0 + x1
            x1 = _rotl(x1, r)
            x1 = x0 ^ x1
        x0 = x0 + jnp.uint32(_KS[(rnd + 1) % 3])
        x1 = x1 + jnp.uint32((_KS[(rnd + 2) % 3] + rnd + 1) & 0xFFFFFFFF)
    return x0 ^ x1


def _horner(coeffs, v):
    p = jnp.float32(coeffs[-1])
    for c in coeffs[-2::-1]:
        p = jnp.float32(c) + p * v
    return p


def _bits_to_normal(bits):
    mant = (bits >> jnp.uint32(9)) | jnp.uint32(0x3F800000)
    f = jax.lax.bitcast_convert_type(mant, jnp.float32) - jnp.float32(1.0)
    lo = jnp.float32(-0.99999994)
    hi = jnp.float32(1.0)
    u = jnp.maximum(lo, f * (hi - lo) + lo)
    t = -jnp.log2(jnp.float32(1.0) - u * u)
    q_central = _horner(_C_CENTRAL, t)
    q_tail = _horner(_C_TAIL, jnp.sqrt(t))
    q = jnp.where(t < jnp.float32(_T_THRESH), q_central, q_tail)
    return u * q


def _tc_main_kernel(o_ref):
    c0 = pl.program_id(0) * _TC_BLOCK
    shape = (_TC_BLOCK, _NROWS)
    cc = jax.lax.broadcasted_iota(jnp.uint32, shape, 0)
    rr = jax.lax.broadcasted_iota(jnp.uint32, shape, 1)
    base = jnp.uint32(c0) + jnp.uint32(_K2)
    x1 = rr * jnp.uint32(_NCOLS) + (cc + base)
    o_ref[...] = _bits_to_normal(_threefry_bits(x1))


def _tc_transform_kernel(_main_ref, bits_ref, o_ref):
    o_ref[...] = _bits_to_normal(bits_ref[...])


def _sc_bits_kernel(out_hbm, scratch, sem):
    wid = lax.axis_index("s") * 2 + lax.axis_index("c")

    @pl.when(wid < _SC_TILES)
    def _():
        row0 = wid * _ROWS_PER_TILE
        lane = lax.iota(jnp.uint32, 16)
        vecs = [(lane + jnp.uint32(16 * j)) * jnp.uint32(_NCOLS)
                for j in range(8)]

        def body(r, carry):
            c = jnp.uint32(_TC_ROWS) + jnp.uint32(row0) + jnp.uint32(r)
            base = c + jnp.uint32(_K2)
            for j in range(8):
                bits = _threefry_bits(vecs[j] + base)
                scratch[r, pl.ds(16 * j, 16)] = bits
            return carry

        lax.fori_loop(0, _ROWS_PER_TILE, body, jnp.uint32(0))
        pltpu.async_copy(
            scratch,
            out_hbm.at[pl.ds(row0, _ROWS_PER_TILE)],
            sem,
        ).wait()


def _sc_bits():
    mesh = plsc.VectorSubcoreMesh(core_axis_name="c", subcore_axis_name="s")
    return pl.kernel(
        _sc_bits_kernel,
        mesh=mesh,
        out_type=jax.ShapeDtypeStruct((_SC_ROWS, _NROWS), jnp.uint32),
        scratch_types=[
            pltpu.VMEM((_ROWS_PER_TILE, _NROWS), jnp.uint32),
            pltpu.SemaphoreType.DMA,
        ],
    )()


@functools.partial(jax.jit, static_argnames=())
def kernel(x):
    nrows, ncols = x.shape
    sc_bits = _sc_bits()

    out_main = pl.pallas_call(
        _tc_main_kernel,
        grid=(_TC_ROWS // _TC_BLOCK,),
        out_specs=pl.BlockSpec((_TC_BLOCK, _NROWS), lambda b: (b, 0)),
        out_shape=jax.ShapeDtypeStruct((_NCOLS, _NROWS), jnp.float32),
        compiler_params=pltpu.CompilerParams(
            dimension_semantics=("arbitrary",),
        ),
    )()

    out_t = pl.pallas_call(
        _tc_transform_kernel,
        grid=(_SC_ROWS // _XF_BLOCK,),
        in_specs=[
            pl.BlockSpec(memory_space=pl.ANY),
            pl.BlockSpec((_XF_BLOCK, _NROWS), lambda b: (b, 0)),
        ],
        out_specs=pl.BlockSpec((_XF_BLOCK, _NROWS),
                               lambda b: (b + _TC_ROWS // _XF_BLOCK, 0)),
        out_shape=jax.ShapeDtypeStruct((_NCOLS, _NROWS), jnp.float32),
        input_output_aliases={0: 0},
        compiler_params=pltpu.CompilerParams(
            dimension_semantics=("arbitrary",),
        ),
    )(out_main, sc_bits)
    return out_t.T
